# baseline (speedup 1.0000x reference)
import functools

import jax
import jax.numpy as jnp
from jax import lax
from jax.experimental import pallas as pl
from jax.experimental.pallas import tpu as pltpu
from jax.experimental.pallas import tpu_sc as plsc

LAM = 0.01
NC = 2
NS = 16
NW = NC * NS
LANES = 16
EC = 4096


def _pass_a(n_nodes, n_graphs, e_pad, e_real):
    per_w = e_pad // NW
    n_groups = per_w // LANES

    @functools.partial(
        pl.kernel,
        out_type=(
            jax.ShapeDtypeStruct((NW, n_nodes), jnp.float32),
            jax.ShapeDtypeStruct((NW, n_graphs), jnp.float32),
        ),
        mesh=plsc.VectorSubcoreMesh(
            core_axis_name="c", subcore_axis_name="s",
            num_cores=NC, num_subcores=NS),
        compiler_params=pltpu.CompilerParams(needs_layout_passes=False),
        scratch_types=[
            pltpu.VMEM((per_w,), jnp.int32),
            pltpu.VMEM((n_nodes,), jnp.int32),
            pltpu.VMEM((n_nodes,), jnp.float32),
            pltpu.VMEM((n_graphs,), jnp.float32),
        ],
    )
    def kern(row_hbm, batch_hbm, deg_out, cnt_out, row_v, batch_v, deg_v, cnt_v):
        wid = lax.axis_index("s") * NC + lax.axis_index("c")
        pltpu.sync_copy(row_hbm.at[pl.ds(wid * per_w, per_w)], row_v)
        pltpu.sync_copy(batch_hbm, batch_v)

        zeros = jnp.zeros((LANES,), jnp.float32)

        def zbody(i, carry):
            deg_v[pl.ds(i * LANES, LANES)] = zeros
            return carry
        lax.fori_loop(0, n_nodes // LANES, zbody, 0)
        for i in range(n_graphs // LANES):
            cnt_v[pl.ds(i * LANES, LANES)] = zeros

        iota = lax.iota(jnp.int32, LANES)
        ones = jnp.ones((LANES,), jnp.float32)
        base = wid * per_w

        def body(g, carry):
            r16 = row_v[pl.ds(g * LANES, LANES)]
            gid = base + g * LANES + iota
            mask = gid < e_real
            plsc.addupdate_scatter(deg_v, [r16], ones, mask=mask)
            be = plsc.load_gather(batch_v, [r16])
            plsc.addupdate_scatter(cnt_v, [be], ones, mask=mask)
            return carry
        lax.fori_loop(0, n_groups, body, 0)

        pltpu.sync_copy(deg_v, deg_out.at[wid])
        pltpu.sync_copy(cnt_v, cnt_out.at[wid])

    return kern


def _pass_mid(n_graphs_ce):
    def body(deg_ref, cnt_ref, batch_ref, logits_ref, labels_ref,
             dis_ref, cinv_ref, ce_ref, ng_ref):
        deg = jnp.sum(deg_ref[...], axis=0, keepdims=True)
        dis_ref[...] = jnp.where(deg > 0.0, lax.rsqrt(deg), 0.0)
        cnt = jnp.sum(cnt_ref[...], axis=0, keepdims=True)
        cinv_ref[...] = 1.0 / jnp.maximum(cnt, 1.0)
        ng_ref[...] = (jnp.max(batch_ref[...], axis=1, keepdims=True)
                       + 1).astype(jnp.float32)
        lg = logits_ref[...]
        m = jnp.max(lg, axis=1, keepdims=True)
        lse = jnp.log(jnp.sum(jnp.exp(lg - m), axis=1, keepdims=True)) + m
        logp = lg - lse
        oh = lax.broadcasted_iota(jnp.int32, lg.shape, 1) == labels_ref[...]
        ce_ref[...] = jnp.sum(-jnp.sum(jnp.where(oh, logp, 0.0),
                                       axis=1, keepdims=True),
                              axis=0, keepdims=True) / n_graphs_ce
    return body


def _pass_w(n_nodes, n_graphs, e_pad):
    per_w = e_pad // NW
    n_groups = per_w // LANES

    @functools.partial(
        pl.kernel,
        out_type=jax.ShapeDtypeStruct((e_pad,), jnp.float32),
        mesh=plsc.VectorSubcoreMesh(
            core_axis_name="c", subcore_axis_name="s",
            num_cores=NC, num_subcores=NS),
        compiler_params=pltpu.CompilerParams(needs_layout_passes=False),
        scratch_types=[
            pltpu.VMEM((per_w,), jnp.int32),
            pltpu.VMEM((per_w,), jnp.int32),
            pltpu.VMEM((per_w,), jnp.float32),
            pltpu.VMEM((n_nodes,), jnp.float32),
            pltpu.VMEM((n_nodes,), jnp.int32),
            pltpu.VMEM((n_graphs,), jnp.float32),
        ],
    )
    def kern(row_hbm, col_hbm, dis_hbm, batch_hbm, cinv_hbm, w_out,
             row_v, col_v, w_v, dis_v, batch_v, cinv_v):
        wid = lax.axis_index("s") * NC + lax.axis_index("c")
        base = wid * per_w
        pltpu.sync_copy(row_hbm.at[pl.ds(base, per_w)], row_v)
        pltpu.sync_copy(col_hbm.at[pl.ds(base, per_w)], col_v)
        pltpu.sync_copy(dis_hbm, dis_v)
        pltpu.sync_copy(batch_hbm, batch_v)
        pltpu.sync_copy(cinv_hbm, cinv_v)

        def body(g, carry):
            r16 = row_v[pl.ds(g * LANES, LANES)]
            c16 = col_v[pl.ds(g * LANES, LANES)]
            dr = plsc.load_gather(dis_v, [r16])
            dc = plsc.load_gather(dis_v, [c16])
            be = plsc.load_gather(batch_v, [r16])
            ci = plsc.load_gather(cinv_v, [be])
            w_v[pl.ds(g * LANES, LANES)] = dr * dc * ci
            return carry
        lax.fori_loop(0, n_groups, body, 0, unroll=4)
        pltpu.sync_copy(w_v, w_out.at[pl.ds(base, per_w)])

    return kern


def _pass_b(n_nodes, n_graphs, n_feat, e_pad):
    fpw = n_feat // NW // 2
    n_chunks = e_pad // EC
    n_pairs = n_chunks // 2
    grp = EC // LANES

    @functools.partial(
        pl.kernel,
        out_type=jax.ShapeDtypeStruct((NW, LANES), jnp.float32),
        mesh=plsc.VectorSubcoreMesh(
            core_axis_name="c", subcore_axis_name="s",
            num_cores=NC, num_subcores=NS),
        compiler_params=pltpu.CompilerParams(needs_layout_passes=False),
        scratch_types=[
            pltpu.VMEM((fpw, n_nodes), jnp.int32),
            pltpu.VMEM((EC,), jnp.int32),
            pltpu.VMEM((EC,), jnp.int32),
            pltpu.VMEM((EC,), jnp.float32),
            pltpu.VMEM((EC,), jnp.int32),
            pltpu.VMEM((EC,), jnp.int32),
            pltpu.VMEM((EC,), jnp.float32),
            pltpu.VMEM((LANES,), jnp.float32),
            pltpu.SemaphoreType.DMA,
            pltpu.SemaphoreType.DMA,
        ],
    )
    def kern(xt_hbm, row_hbm, col_hbm, w_hbm, part_out,
             slab, r0, c0, w0, r1, c1, w1, out_v, sem0, sem1):
        cid = lax.axis_index("c")
        sid = lax.axis_index("s")
        wid = sid * NC + cid
        pltpu.sync_copy(xt_hbm.at[pl.ds(wid * fpw, fpw)], slab)

        def issue(c, rb, cb, wb, sem):
            pltpu.async_copy(row_hbm.at[pl.ds(c * EC, EC)], rb, sem)
            pltpu.async_copy(col_hbm.at[pl.ds(c * EC, EC)], cb, sem)
            pltpu.async_copy(w_hbm.at[pl.ds(c * EC, EC)], wb, sem)

        def drain(rb, cb, wb, sem):
            dummy = row_hbm.at[pl.ds(0, EC)]
            pltpu.make_async_copy(dummy, rb, sem).wait()
            pltpu.make_async_copy(dummy, cb, sem).wait()
            pltpu.make_async_copy(w_hbm.at[pl.ds(0, EC)], wb, sem).wait()

        fvecs = [jnp.full((LANES,), f, jnp.int32) for f in range(fpw)]
        zero = jnp.zeros((LANES,), jnp.float32)

        def chunk(rb, cb, wb, tot):
            def gbody(g, tot):
                r16 = rb[pl.ds(g * LANES, LANES)]
                c16 = cb[pl.ds(g * LANES, LANES)]
                w16 = wb[pl.ds(g * LANES, LANES)]
                acc = zero
                for f in range(fpw):
                    vr = plsc.load_gather(slab, [fvecs[f], r16])
                    vc = plsc.load_gather(slab, [fvecs[f], c16])
                    ur0, ur1 = plsc.unpack(
                        plsc.bitcast(vr, jnp.bfloat16),
                        format=plsc.PackFormat.INTERLEAVED)
                    uc0, uc1 = plsc.unpack(
                        plsc.bitcast(vc, jnp.bfloat16),
                        format=plsc.PackFormat.INTERLEAVED)
                    da = ur0 - uc0
                    db = ur1 - uc1
                    acc = acc + da * da + db * db
                return tot + acc * w16
            return lax.fori_loop(0, grp, gbody, tot, unroll=8)

        issue(0, r0, c0, w0, sem0)
        issue(1, r1, c1, w1, sem1)

        def pair(j, tot):
            drain(r0, c0, w0, sem0)
            tot = chunk(r0, c0, w0, tot)

            @pl.when(j < n_pairs - 1)
            def _():
                issue(2 * j + 2, r0, c0, w0, sem0)

            drain(r1, c1, w1, sem1)
            tot = chunk(r1, c1, w1, tot)

            @pl.when(j < n_pairs - 1)
            def _():
                issue(2 * j + 3, r1, c1, w1, sem1)
            return tot

        tot = lax.fori_loop(0, n_pairs, pair, zero)
        out_v[...] = tot
        pltpu.sync_copy(out_v, part_out.at[wid])

    return kern


def _combine_body(part_ref, ce_ref, ng_ref, out_ref):
    s = jnp.sum(jnp.sum(part_ref[...], axis=1, keepdims=True),
                axis=0, keepdims=True)
    out_ref[...] = ce_ref[...] + LAM * (s / ng_ref[...])


@jax.jit
def kernel(logits, labels, x, edge_index, batch):
    n_nodes, n_feat = x.shape
    n_ce, n_cls = logits.shape
    n_graphs = 128
    e_real = edge_index.shape[1]
    block = EC * 2
    e_pad = ((e_real + block - 1) // block) * block

    row = jnp.pad(edge_index[0], (0, e_pad - e_real))
    col = jnp.pad(edge_index[1], (0, e_pad - e_real))
    xt = jax.lax.bitcast_convert_type(
        x.astype(jnp.bfloat16).reshape(n_nodes, n_feat // 2, 2)
        .transpose(1, 0, 2),
        jnp.int32)

    deg_part, cnt_part = _pass_a(n_nodes, n_graphs, e_pad, e_real)(row, batch)

    dis2, cinv2, ce, ng = pl.pallas_call(
        _pass_mid(float(n_ce)),
        out_shape=(
            jax.ShapeDtypeStruct((1, n_nodes), jnp.float32),
            jax.ShapeDtypeStruct((1, n_graphs), jnp.float32),
            jax.ShapeDtypeStruct((1, 1), jnp.float32),
            jax.ShapeDtypeStruct((1, 1), jnp.float32),
        ),
    )(deg_part, cnt_part, batch.reshape(1, n_nodes), logits,
      labels.reshape(n_ce, 1))

    w = _pass_w(n_nodes, n_graphs, e_pad)(
        row, col, dis2.reshape(n_nodes), batch, cinv2.reshape(n_graphs))
    part = _pass_b(n_nodes, n_graphs, n_feat, e_pad)(xt, row, col, w)

    out = pl.pallas_call(
        _combine_body,
        out_shape=jax.ShapeDtypeStruct((1, 1), jnp.float32),
    )(part, ce, ng)
    return out[0, 0]


# speedup vs baseline: 1.7074x; 1.7074x over previous
import functools

import jax
import jax.numpy as jnp
from jax import lax
from jax.experimental import pallas as pl
from jax.experimental.pallas import tpu as pltpu
from jax.experimental.pallas import tpu_sc as plsc

LAM = 0.01
NC = 2
NS = 16
NW = NC * NS
LANES = 16
EC = 4096


def _pass_a(n_nodes, n_graphs, e_pad, e_real):
    per_w = e_pad // NW
    n_groups = per_w // LANES

    @functools.partial(
        pl.kernel,
        out_type=(
            jax.ShapeDtypeStruct((NW, n_nodes), jnp.float32),
            jax.ShapeDtypeStruct((NW, n_graphs), jnp.float32),
        ),
        mesh=plsc.VectorSubcoreMesh(
            core_axis_name="c", subcore_axis_name="s",
            num_cores=NC, num_subcores=NS),
        compiler_params=pltpu.CompilerParams(needs_layout_passes=False),
        scratch_types=[
            pltpu.VMEM((per_w,), jnp.int32),
            pltpu.VMEM((n_nodes,), jnp.int32),
            pltpu.VMEM((n_nodes,), jnp.float32),
            pltpu.VMEM((n_graphs,), jnp.float32),
        ],
    )
    def kern(row_hbm, batch_hbm, deg_out, cnt_out, row_v, batch_v, deg_v, cnt_v):
        wid = lax.axis_index("s") * NC + lax.axis_index("c")
        pltpu.sync_copy(row_hbm.at[pl.ds(wid * per_w, per_w)], row_v)
        pltpu.sync_copy(batch_hbm, batch_v)

        zeros = jnp.zeros((LANES,), jnp.float32)

        def zbody(i, carry):
            deg_v[pl.ds(i * LANES, LANES)] = zeros
            return carry
        lax.fori_loop(0, n_nodes // LANES, zbody, 0)
        for i in range(n_graphs // LANES):
            cnt_v[pl.ds(i * LANES, LANES)] = zeros

        iota = lax.iota(jnp.int32, LANES)
        ones = jnp.ones((LANES,), jnp.float32)
        base = wid * per_w

        def body(g, carry):
            r16 = row_v[pl.ds(g * LANES, LANES)]
            gid = base + g * LANES + iota
            mask = gid < e_real
            plsc.addupdate_scatter(deg_v, [r16], ones, mask=mask)
            be = plsc.load_gather(batch_v, [r16])
            plsc.addupdate_scatter(cnt_v, [be], ones, mask=mask)
            return carry
        lax.fori_loop(0, n_groups, body, 0)

        pltpu.sync_copy(deg_v, deg_out.at[wid])
        pltpu.sync_copy(cnt_v, cnt_out.at[wid])

    return kern


def _pass_mid(n_graphs_ce):
    def body(deg_ref, cnt_ref, batch_ref, logits_ref, labels_ref,
             dis_ref, cinv_ref, ce_ref, ng_ref):
        deg = jnp.sum(deg_ref[...], axis=0, keepdims=True)
        dis_ref[...] = jnp.where(deg > 0.0, lax.rsqrt(deg), 0.0)
        cnt = jnp.sum(cnt_ref[...], axis=0, keepdims=True)
        cinv_ref[...] = 1.0 / jnp.maximum(cnt, 1.0)
        ng_ref[...] = (jnp.max(batch_ref[...], axis=1, keepdims=True)
                       + 1).astype(jnp.float32)
        lg = logits_ref[...]
        m = jnp.max(lg, axis=1, keepdims=True)
        lse = jnp.log(jnp.sum(jnp.exp(lg - m), axis=1, keepdims=True)) + m
        logp = lg - lse
        oh = lax.broadcasted_iota(jnp.int32, lg.shape, 1) == labels_ref[...]
        ce_ref[...] = jnp.sum(-jnp.sum(jnp.where(oh, logp, 0.0),
                                       axis=1, keepdims=True),
                              axis=0, keepdims=True) / n_graphs_ce
    return body


def _pass_w(n_nodes, n_graphs, e_pad):
    per_w = e_pad // NW
    n_groups = per_w // LANES

    @functools.partial(
        pl.kernel,
        out_type=jax.ShapeDtypeStruct((e_pad,), jnp.float32),
        mesh=plsc.VectorSubcoreMesh(
            core_axis_name="c", subcore_axis_name="s",
            num_cores=NC, num_subcores=NS),
        compiler_params=pltpu.CompilerParams(needs_layout_passes=False),
        scratch_types=[
            pltpu.VMEM((per_w,), jnp.int32),
            pltpu.VMEM((per_w,), jnp.int32),
            pltpu.VMEM((per_w,), jnp.float32),
            pltpu.VMEM((n_nodes,), jnp.float32),
            pltpu.VMEM((n_nodes,), jnp.int32),
            pltpu.VMEM((n_graphs,), jnp.float32),
        ],
    )
    def kern(row_hbm, col_hbm, dis_hbm, batch_hbm, cinv_hbm, w_out,
             row_v, col_v, w_v, dis_v, batch_v, cinv_v):
        wid = lax.axis_index("s") * NC + lax.axis_index("c")
        base = wid * per_w
        pltpu.sync_copy(row_hbm.at[pl.ds(base, per_w)], row_v)
        pltpu.sync_copy(col_hbm.at[pl.ds(base, per_w)], col_v)
        pltpu.sync_copy(dis_hbm, dis_v)
        pltpu.sync_copy(batch_hbm, batch_v)
        pltpu.sync_copy(cinv_hbm, cinv_v)

        def body(g, carry):
            r16 = row_v[pl.ds(g * LANES, LANES)]
            c16 = col_v[pl.ds(g * LANES, LANES)]
            dr = plsc.load_gather(dis_v, [r16])
            dc = plsc.load_gather(dis_v, [c16])
            be = plsc.load_gather(batch_v, [r16])
            ci = plsc.load_gather(cinv_v, [be])
            w_v[pl.ds(g * LANES, LANES)] = dr * dc * ci
            return carry
        lax.fori_loop(0, n_groups, body, 0, unroll=4)
        pltpu.sync_copy(w_v, w_out.at[pl.ds(base, per_w)])

    return kern


def _pass_b(n_nodes, n_graphs, n_feat, e_pad):
    fpw = n_feat // NW // 2
    n_chunks = e_pad // EC
    n_pairs = n_chunks // 2
    grp = EC // LANES

    @functools.partial(
        pl.kernel,
        out_type=jax.ShapeDtypeStruct((NW, LANES), jnp.float32),
        mesh=plsc.VectorSubcoreMesh(
            core_axis_name="c", subcore_axis_name="s",
            num_cores=NC, num_subcores=NS),
        compiler_params=pltpu.CompilerParams(needs_layout_passes=False),
        scratch_types=[
            pltpu.VMEM((fpw, n_nodes), jnp.int32),
            pltpu.VMEM((EC,), jnp.int32),
            pltpu.VMEM((EC,), jnp.int32),
            pltpu.VMEM((EC,), jnp.float32),
            pltpu.VMEM((EC,), jnp.int32),
            pltpu.VMEM((EC,), jnp.int32),
            pltpu.VMEM((EC,), jnp.float32),
            pltpu.VMEM((LANES,), jnp.float32),
            pltpu.SemaphoreType.DMA,
            pltpu.SemaphoreType.DMA,
        ],
    )
    def kern(xt_hbm, row_hbm, col_hbm, w_hbm, part_out,
             slab, r0, c0, w0, r1, c1, w1, out_v, sem0, sem1):
        cid = lax.axis_index("c")
        sid = lax.axis_index("s")
        wid = sid * NC + cid
        pltpu.sync_copy(xt_hbm.at[pl.ds(wid * fpw, fpw)], slab)

        def issue(c, rb, cb, wb, sem):
            pltpu.async_copy(row_hbm.at[pl.ds(c * EC, EC)], rb, sem)
            pltpu.async_copy(col_hbm.at[pl.ds(c * EC, EC)], cb, sem)
            pltpu.async_copy(w_hbm.at[pl.ds(c * EC, EC)], wb, sem)

        def drain(rb, cb, wb, sem):
            dummy = row_hbm.at[pl.ds(0, EC)]
            pltpu.make_async_copy(dummy, rb, sem).wait()
            pltpu.make_async_copy(dummy, cb, sem).wait()
            pltpu.make_async_copy(w_hbm.at[pl.ds(0, EC)], wb, sem).wait()

        fvecs = [jnp.full((LANES,), f, jnp.int32) for f in range(fpw)]
        zero = jnp.zeros((LANES,), jnp.float32)

        def chunk(rb, cb, wb, tot):
            def gbody(g, tot):
                r16 = rb[pl.ds(g * LANES, LANES)]
                c16 = cb[pl.ds(g * LANES, LANES)]
                w16 = wb[pl.ds(g * LANES, LANES)]
                acc = zero
                for f in range(fpw):
                    vr = plsc.load_gather(slab, [fvecs[f], r16])
                    vc = plsc.load_gather(slab, [fvecs[f], c16])
                    ur0, ur1 = plsc.unpack(
                        plsc.bitcast(vr, jnp.bfloat16),
                        format=plsc.PackFormat.INTERLEAVED)
                    uc0, uc1 = plsc.unpack(
                        plsc.bitcast(vc, jnp.bfloat16),
                        format=plsc.PackFormat.INTERLEAVED)
                    da = ur0 - uc0
                    db = ur1 - uc1
                    acc = acc + da * da + db * db
                return tot + acc * w16
            return lax.fori_loop(0, grp, gbody, tot, unroll=4)

        issue(0, r0, c0, w0, sem0)
        issue(1, r1, c1, w1, sem1)

        def pair(j, tot):
            drain(r0, c0, w0, sem0)
            tot = chunk(r0, c0, w0, tot)

            @pl.when(j < n_pairs - 1)
            def _():
                issue(2 * j + 2, r0, c0, w0, sem0)

            drain(r1, c1, w1, sem1)
            tot = chunk(r1, c1, w1, tot)

            @pl.when(j < n_pairs - 1)
            def _():
                issue(2 * j + 3, r1, c1, w1, sem1)
            return tot

        tot = lax.fori_loop(0, n_pairs, pair, zero)
        out_v[...] = tot
        pltpu.sync_copy(out_v, part_out.at[wid])

    return kern


def _combine_body(part_ref, ce_ref, ng_ref, out_ref):
    s = jnp.sum(jnp.sum(part_ref[...], axis=1, keepdims=True),
                axis=0, keepdims=True)
    out_ref[...] = ce_ref[...] + LAM * (s / ng_ref[...])


@jax.jit
def kernel(logits, labels, x, edge_index, batch):
    n_nodes, n_feat = x.shape
    n_ce, n_cls = logits.shape
    n_graphs = 128
    e_real = edge_index.shape[1]
    block = EC * 2
    e_pad = ((e_real + block - 1) // block) * block

    row = jnp.pad(edge_index[0], (0, e_pad - e_real))
    col = jnp.pad(edge_index[1], (0, e_pad - e_real))
    xt = jax.lax.bitcast_convert_type(
        x.astype(jnp.bfloat16).reshape(n_nodes, n_feat // 2, 2)
        .transpose(1, 0, 2),
        jnp.int32)

    deg_part, cnt_part = _pass_a(n_nodes, n_graphs, e_pad, e_real)(row, batch)

    dis2, cinv2, ce, ng = pl.pallas_call(
        _pass_mid(float(n_ce)),
        out_shape=(
            jax.ShapeDtypeStruct((1, n_nodes), jnp.float32),
            jax.ShapeDtypeStruct((1, n_graphs), jnp.float32),
            jax.ShapeDtypeStruct((1, 1), jnp.float32),
            jax.ShapeDtypeStruct((1, 1), jnp.float32),
        ),
    )(deg_part, cnt_part, batch.reshape(1, n_nodes), logits,
      labels.reshape(n_ce, 1))

    w = _pass_w(n_nodes, n_graphs, e_pad)(
        row, col, dis2.reshape(n_nodes), batch, cinv2.reshape(n_graphs))
    part = _pass_b(n_nodes, n_graphs, n_feat, e_pad)(xt, row, col, w)

    out = pl.pallas_call(
        _combine_body,
        out_shape=jax.ShapeDtypeStruct((1, 1), jnp.float32),
    )(part, ce, ng)
    return out[0, 0]
